# baseline (speedup 1.0000x reference)
import functools

import jax
import jax.numpy as jnp
from jax import lax
from jax.experimental import pallas as pl
from jax.experimental.pallas import tpu as pltpu
from jax.experimental.pallas import tpu_sc as plsc

BATCH = 16384
DIM = 32
NUM_CORES = 2
NUM_SUBCORES = 16
NW = NUM_CORES * NUM_SUBCORES
BPW = BATCH // NW
CHUNK = 128
NCHUNK = BPW // CHUNK


def _distmult_kernel(entity_hbm, relation_hbm, h_hbm, t_hbm, r_hbm, out_hbm,
                     hi_v, ti_v, ri_v, h_v, t_v, r_v, out_v, sem):
    wid = lax.axis_index("s") * NUM_CORES + lax.axis_index("c")
    base = wid * BPW

    for j in range(NCHUNK):
        pltpu.sync_copy(h_hbm.at[pl.ds(base + j * CHUNK, CHUNK)], hi_v.at[j])
        pltpu.sync_copy(t_hbm.at[pl.ds(base + j * CHUNK, CHUNK)], ti_v.at[j])
        pltpu.sync_copy(r_hbm.at[pl.ds(base + j * CHUNK, CHUNK)], ri_v.at[j])

    copies = []
    for j in range(NCHUNK):
        copies.append(pltpu.async_copy(
            entity_hbm.at[hi_v.at[j]], h_v.at[pl.ds(j * CHUNK, CHUNK)], sem))
        copies.append(pltpu.async_copy(
            entity_hbm.at[ti_v.at[j]], t_v.at[pl.ds(j * CHUNK, CHUNK)], sem))
        copies.append(pltpu.async_copy(
            relation_hbm.at[ri_v.at[j]], r_v.at[pl.ds(j * CHUNK, CHUNK)], sem))
    for c in copies:
        c.wait()

    lane = lax.iota(jnp.int32, 16)

    def body(blk, carry):
        vec = jnp.zeros((16,), jnp.float32)
        for i in range(16):
            row = blk * 16 + i
            h0 = h_v[row, pl.ds(0, 16)]
            h1 = h_v[row, pl.ds(16, 16)]
            r0 = r_v[row, pl.ds(0, 16)]
            r1 = r_v[row, pl.ds(16, 16)]
            t0 = t_v[row, pl.ds(0, 16)]
            t1 = t_v[row, pl.ds(16, 16)]
            acc = h0 * r0 * t0 + h1 * r1 * t1
            vec = jnp.where(lane == i, jnp.sum(acc), vec)
        out_v[pl.ds(blk * 16, 16)] = vec
        return carry

    lax.fori_loop(0, BPW // 16, body, 0)

    pltpu.sync_copy(out_v, out_hbm.at[pl.ds(base, BPW)])


def kernel(entity, relation, h_index, t_index, r_index):
    k = functools.partial(
        pl.kernel,
        mesh=plsc.VectorSubcoreMesh(core_axis_name="c", subcore_axis_name="s"),
        out_type=jax.ShapeDtypeStruct((BATCH,), jnp.float32),
        compiler_params=pltpu.CompilerParams(
            needs_layout_passes=False, use_tc_tiling_on_sc=False),
        scratch_types=[
            pltpu.VMEM((NCHUNK, CHUNK), jnp.int32),
            pltpu.VMEM((NCHUNK, CHUNK), jnp.int32),
            pltpu.VMEM((NCHUNK, CHUNK), jnp.int32),
            pltpu.VMEM((BPW, DIM), jnp.float32),
            pltpu.VMEM((BPW, DIM), jnp.float32),
            pltpu.VMEM((BPW, DIM), jnp.float32),
            pltpu.VMEM((BPW,), jnp.float32),
            pltpu.SemaphoreType.DMA,
        ],
    )(_distmult_kernel)
    return k(entity, relation,
             h_index.astype(jnp.int32), t_index.astype(jnp.int32),
             r_index.astype(jnp.int32))
